# baseline (speedup 1.0000x reference)
import functools

import jax
import jax.numpy as jnp
from jax import lax
from jax.experimental import pallas as pl
from jax.experimental.pallas import tpu as pltpu
from jax.experimental.pallas import tpu_sc as plsc

VOCAB = 1000
VPAD = 1024
KTILES = VPAD // 128
TAILK = KTILES - 1
TAILW = VOCAB - TAILK * 128
BT = 51200
NC, NS = 2, 16
NW = NC * NS
PER_TILE = BT // NW
CHUNK = 32
NCHUNK = PER_TILE // CHUNK
CIDX_PER_TILE = PER_TILE * KTILES


def _row_logsumexp(table):
    def body(t_ref, o_ref):
        x = t_ref[...]
        m = jnp.max(x, axis=1, keepdims=True)
        s = jnp.sum(jnp.exp(x - m), axis=1, keepdims=True)
        o_ref[...] = jnp.log(s) + m

    return pl.pallas_call(
        body,
        out_shape=jax.ShapeDtypeStruct((VOCAB, 1), jnp.float32),
    )(table)


def _sc_gather_and_loss(idx_flat, tgt_flat, cidx, table_c, table_flat, lse):
    mesh = plsc.VectorSubcoreMesh(core_axis_name="c", subcore_axis_name="s")

    @functools.partial(
        pl.kernel,
        mesh=mesh,
        out_type=[
            jax.ShapeDtypeStruct((BT, VOCAB), jnp.float32),
            jax.ShapeDtypeStruct((NW, 16), jnp.float32),
        ],
        scratch_types=[
            pltpu.VMEM((PER_TILE,), jnp.int32),
            pltpu.VMEM((PER_TILE,), jnp.int32),
            pltpu.VMEM((CIDX_PER_TILE,), jnp.int32),
            pltpu.VMEM((CHUNK, VOCAB), jnp.float32),
            pltpu.VMEM((CHUNK, VOCAB), jnp.float32),
            pltpu.VMEM((CHUNK, 128), jnp.float32),
            pltpu.VMEM((CHUNK, 128), jnp.float32),
            pltpu.VMEM((PER_TILE,), jnp.float32),
            pltpu.VMEM((PER_TILE,), jnp.float32),
            pltpu.VMEM((16,), jnp.float32),
            pltpu.SemaphoreType.DMA,
            pltpu.SemaphoreType.DMA,
            pltpu.SemaphoreType.DMA,
            pltpu.SemaphoreType.DMA,
            pltpu.SemaphoreType.DMA,
        ],
    )
    def k(idx_hbm, tgt_hbm, cidx_hbm, tc_hbm, tflat_hbm, lse_hbm,
          out_hbm, part_hbm,
          idx_v, tgt_v, cidx_v, rows0_v, rows1_v, tail0_v, tail1_v,
          tl_v, ls_v, acc_v,
          sem_g0, sem_g1, sem_s0, sem_s1, sem_e):
        rows = (rows0_v, rows1_v)
        tails = (tail0_v, tail1_v)
        sem_g = (sem_g0, sem_g1)
        sem_s = (sem_s0, sem_s1)
        wid = lax.axis_index("s") * NC + lax.axis_index("c")
        base = wid * PER_TILE
        pltpu.sync_copy(idx_hbm.at[pl.ds(base, PER_TILE)], idx_v)
        pltpu.sync_copy(tgt_hbm.at[pl.ds(base, PER_TILE)], tgt_v)
        pltpu.sync_copy(
            cidx_hbm.at[pl.ds(base * KTILES, CIDX_PER_TILE)], cidx_v
        )

        def chunk_gathers(c, b, start):
            cps = []
            for kk in range(KTILES):
                src = tc_hbm.at[
                    cidx_v.at[pl.ds(c * CHUNK * KTILES + kk * CHUNK, CHUNK)]
                ]
                if kk == TAILK:
                    dst = tails[b]
                else:
                    dst = rows[b].at[pl.ds(0, CHUNK), pl.ds(kk * 128, 128)]
                if start:
                    cps.append(pltpu.async_copy(src, dst, sem_g[b]))
                else:
                    cps.append(pltpu.make_async_copy(src, dst, sem_g[b]))
            return cps

        chunk_gathers(0, 0, start=True)
        chunk_gathers(1, 1, start=True)

        def pair_body(p, acc):
            for b in range(2):
                c = 2 * p + b
                o = c * CHUNK
                elem_cps = []
                for g in range(CHUNK // 16):
                    og = o + g * 16
                    idx16 = idx_v[pl.ds(og, 16)]
                    t16 = tgt_v[pl.ds(og, 16)]
                    flat16 = idx16 * VOCAB + t16
                    elem_cps.append(pltpu.async_copy(
                        tflat_hbm.at[flat16], tl_v.at[pl.ds(og, 16)], sem_e
                    ))
                    elem_cps.append(pltpu.async_copy(
                        lse_hbm.at[idx16], ls_v.at[pl.ds(og, 16)], sem_e
                    ))
                for cp in chunk_gathers(c, b, start=False):
                    cp.wait()
                for r in range(CHUNK):
                    for off in (0, 16, 32, 48, 64, TAILW - 16, 80):
                        rows[b][r, pl.ds(TAILK * 128 + off, 16)] = (
                            tails[b][r, pl.ds(off, 16)]
                        )
                scat = pltpu.async_copy(
                    rows[b],
                    out_hbm.at[pl.ds(base + o, CHUNK)],
                    sem_s[b],
                )
                for cp in elem_cps:
                    cp.wait()
                for g in range(CHUNK // 16):
                    og = o + g * 16
                    acc = acc + (ls_v[pl.ds(og, 16)] - tl_v[pl.ds(og, 16)])
                scat.wait()
                @pl.when(c + 2 < NCHUNK)
                def _():
                    chunk_gathers_dyn(c + 2, b)
            return acc

        def chunk_gathers_dyn(c, b):
            for kk in range(KTILES):
                src = tc_hbm.at[
                    cidx_v.at[pl.ds(c * CHUNK * KTILES + kk * CHUNK, CHUNK)]
                ]
                if kk == TAILK:
                    dst = tails[b]
                else:
                    dst = rows[b].at[pl.ds(0, CHUNK), pl.ds(kk * 128, 128)]
                pltpu.async_copy(src, dst, sem_g[b])

        acc = lax.fori_loop(
            0, NCHUNK // 2, pair_body, jnp.zeros((16,), jnp.float32)
        )
        acc_v[...] = acc * (1.0 / BT)
        pltpu.sync_copy(acc_v, part_hbm.at[wid])

    return k(idx_flat, tgt_flat, cidx, table_c, table_flat, lse)


def kernel(index, targets, token_embedding_table):
    idx_flat = jnp.transpose(index).reshape(-1)
    tgt_flat = targets.reshape(-1)
    lse = _row_logsumexp(token_embedding_table).reshape(VOCAB)
    table_c = jnp.pad(
        token_embedding_table, ((0, 0), (0, VPAD - VOCAB))
    ).reshape(VOCAB * KTILES, 128)
    idx_chunks = idx_flat.reshape(BT // CHUNK, CHUNK)
    cidx = (
        idx_chunks[:, None, :] * KTILES
        + jnp.arange(KTILES, dtype=jnp.int32)[None, :, None]
    ).reshape(-1)
    logits, part = _sc_gather_and_loss(
        idx_flat, tgt_flat, cidx, table_c,
        token_embedding_table.reshape(-1), lse
    )
    loss = jnp.sum(part)
    return (logits, loss)
